# Initial kernel scaffold; baseline (speedup 1.0000x reference)
import jax
import jax.numpy as jnp
from jax.experimental import pallas as pl


def kernel(x, connection_indices, interpolation_weights):
    raise NotImplementedError("write your pallas kernel here")


# speedup vs baseline: 5.7280x; 5.7280x over previous
import functools

import jax
import jax.numpy as jnp
from jax import lax
from jax.experimental import pallas as pl
from jax.experimental.pallas import tpu as pltpu
from jax.experimental.pallas import tpu_sc as plsc

_B = 4
_N_IN = 12288
_N_OUT = 49152
_K = 4
_C = 32
_NC = 2
_NS = 16
_NW = _NC * _NS
_RPW = _N_OUT // _NW
_CHUNK = 32
_NSTEP = _RPW // _CHUNK
_G = _CHUNK * _K
_BC = _B * _C


def _sc_body(xt, ci, wf, out, idx_v, g_v, w_v, o_v, sem):
    wid = lax.axis_index("s") * _NC + lax.axis_index("c")
    base0 = wid * _RPW

    def step(s, carry):
        rbase = base0 + s * _CHUNK
        pltpu.sync_copy(ci.at[pl.ds(rbase * _K, _G)], idx_v)
        pltpu.sync_copy(wf.at[pl.ds(rbase * _K, _G)], w_v)
        pltpu.async_copy(xt.at[idx_v], g_v, sem).wait()

        def row4(i, c2):
            wvec = w_v[pl.ds(i * 16, 16)]
            for j in range(4):
                r = i * 4 + j
                b4 = r * _K
                ws = [wvec[j * _K + k] for k in range(_K)]
                for b in range(_B):
                    for h in range(2):
                        col = b * _C + h * 16
                        acc = ws[0] * g_v[b4, pl.ds(col, 16)]
                        for k in range(1, _K):
                            acc = acc + ws[k] * g_v[b4 + k, pl.ds(col, 16)]
                        o_v[b, r, pl.ds(h * 16, 16)] = acc
            return c2

        lax.fori_loop(0, _CHUNK // 4, row4, 0)
        for b in range(_B):
            pltpu.sync_copy(o_v.at[b], out.at[b, pl.ds(rbase, _CHUNK)])
        return carry

    lax.fori_loop(0, _NSTEP, step, 0)


_upsample = functools.partial(
    pl.kernel,
    out_type=jax.ShapeDtypeStruct((_B, _N_OUT, _C), jnp.float32),
    mesh=plsc.VectorSubcoreMesh(core_axis_name="c", subcore_axis_name="s"),
    scratch_types=[
        pltpu.VMEM((_G,), jnp.int32),
        pltpu.VMEM((_G, _BC), jnp.float32),
        pltpu.VMEM((_G,), jnp.float32),
        pltpu.VMEM((_B, _CHUNK, _C), jnp.float32),
        pltpu.SemaphoreType.DMA,
    ],
)(_sc_body)


def kernel(x, connection_indices, interpolation_weights):
    xt = jnp.transpose(x, (1, 0, 2)).reshape(_N_IN, _BC)
    ci = connection_indices.reshape(-1)
    wf = interpolation_weights.reshape(-1)
    return _upsample(xt, ci, wf)


# speedup vs baseline: 7.9469x; 1.3874x over previous
import functools

import jax
import jax.numpy as jnp
from jax import lax
from jax.experimental import pallas as pl
from jax.experimental.pallas import tpu as pltpu
from jax.experimental.pallas import tpu_sc as plsc

_B = 4
_N_IN = 12288
_N_OUT = 49152
_K = 4
_C = 32
_NC = 2
_NS = 16
_NW = _NC * _NS
_RPW = _N_OUT // _NW
_CHUNK = 64
_NSTEP = _RPW // _CHUNK
_G = _CHUNK * _K
_GR = 128
_NGS = _G // _GR
_IROWS = _RPW * _K // _GR
_BC = _B * _C
_ORPS = _CHUNK * _C // 128


def _sc_body(xt, ci2, wf, out, idx_all, w_all, g_v, o_v, gsem, osem):
    wid = lax.axis_index("s") * _NC + lax.axis_index("c")
    base0 = wid * _RPW

    pltpu.sync_copy(ci2.at[pl.ds(wid * _IROWS, _IROWS)], idx_all)
    pltpu.sync_copy(wf.at[pl.ds(base0 * _K, _RPW * _K)], w_all)

    def start_gather(s, bi):
        for j in range(_NGS):
            pltpu.async_copy(
                xt.at[idx_all.at[s * _NGS + j]],
                g_v.at[bi, pl.ds(j * _GR, _GR)],
                gsem.at[bi],
            )

    start_gather(0, 0)
    start_gather(1, 1)

    def compute_step(s, bi):
        woff = s * (_CHUNK * _K)

        def row4(i, c2):
            wvec = w_all[pl.ds(woff + i * 16, 16)]
            for j in range(4):
                r = i * 4 + j
                b4 = r * _K
                ws = [wvec[j * _K + k] for k in range(_K)]
                for b in range(_B):
                    for h in range(2):
                        col = b * _C + h * 16
                        acc = ws[0] * g_v[bi, b4, pl.ds(col, 16)]
                        for k in range(1, _K):
                            acc = acc + ws[k] * g_v[bi, b4 + k, pl.ds(col, 16)]
                        o_v[bi, b, i, pl.ds(j * _C + h * 16, 16)] = acc
            return c2

        lax.fori_loop(0, _CHUNK // 4, row4, 0)

    def sbody(s, carry):
        bi = lax.rem(s, 2)
        rbase = base0 + s * _CHUNK
        pltpu.make_async_copy(
            xt.at[pl.ds(0, _G)], g_v.at[bi], gsem.at[bi]
        ).wait()
        rbase4 = pl.multiple_of(rbase * _C // 128, _ORPS)

        @pl.when(s >= 2)
        def _():
            for b in range(_B):
                pltpu.make_async_copy(
                    o_v.at[bi, b], out.at[b, pl.ds(rbase4, _ORPS)], osem.at[bi]
                ).wait()

        compute_step(s, bi)

        @pl.when(s + 2 < _NSTEP)
        def _():
            start_gather(s + 2, bi)

        for b in range(_B):
            pltpu.async_copy(
                o_v.at[bi, b], out.at[b, pl.ds(rbase4, _ORPS)], osem.at[bi]
            )
        return carry

    lax.fori_loop(0, _NSTEP, sbody, 0)

    for sl in (_NSTEP - 2, _NSTEP - 1):
        bi = sl % 2
        rb4 = pl.multiple_of((base0 + sl * _CHUNK) * _C // 128, _ORPS)
        for b in range(_B):
            pltpu.make_async_copy(
                o_v.at[bi, b], out.at[b, pl.ds(rb4, _ORPS)], osem.at[bi]
            ).wait()


_upsample = functools.partial(
    pl.kernel,
    out_type=jax.ShapeDtypeStruct((_B, _N_OUT * _C // 128, 128), jnp.float32),
    mesh=plsc.VectorSubcoreMesh(core_axis_name="c", subcore_axis_name="s"),
    scratch_types=[
        pltpu.VMEM((_IROWS, _GR), jnp.int32),
        pltpu.VMEM((_RPW * _K,), jnp.float32),
        pltpu.VMEM((2, _G, _BC), jnp.float32),
        pltpu.VMEM((2, _B, _CHUNK // 4, 128), jnp.float32),
        pltpu.SemaphoreType.DMA((2,)),
        pltpu.SemaphoreType.DMA((2,)),
    ],
)(_sc_body)


def kernel(x, connection_indices, interpolation_weights):
    xt = jnp.transpose(x, (1, 0, 2)).reshape(_N_IN, _BC)
    ci2 = connection_indices.reshape(_N_OUT * _K // _GR, _GR)
    wf = interpolation_weights.reshape(-1)
    return _upsample(xt, ci2, wf).reshape(_B, _N_OUT, _C)


# speedup vs baseline: 9.5879x; 1.2065x over previous
import functools

import jax
import jax.numpy as jnp
from jax import lax
from jax.experimental import pallas as pl
from jax.experimental.pallas import tpu as pltpu
from jax.experimental.pallas import tpu_sc as plsc

_B = 4
_N_IN = 12288
_N_OUT = 49152
_K = 4
_C = 32
_NC = 2
_NS = 16
_NW = _NC * _NS
_RPW = _N_OUT // _NW
_CHUNK = 32
_NSTEP = _RPW // _CHUNK
_G = _CHUNK * _K
_IROWS = _RPW * _K // _G
_BC = _B * _C
_NBUF = 3


def _sc_body(xt, ci2, wf, out, idx_all, w_all, g_v, o_v, gsem, osem):
    wid = lax.axis_index("s") * _NC + lax.axis_index("c")
    base0 = wid * _RPW

    pltpu.sync_copy(ci2.at[pl.ds(wid * _IROWS, _IROWS)], idx_all)
    pltpu.sync_copy(wf.at[pl.ds(base0 * _K, _RPW * _K)], w_all)

    def start_gather(s, bi):
        pltpu.async_copy(xt.at[idx_all.at[s]], g_v.at[bi], gsem.at[bi])

    start_gather(0, 0)
    start_gather(1, 1)

    def compute_step(s, bi):
        woff = s * (_CHUNK * _K)
        for i in range(_CHUNK // 4):
            wvec = w_all[pl.ds(woff + i * 16, 16)]
            for j in range(4):
                r = i * 4 + j
                b4 = r * _K
                ws = [wvec[j * _K + k] for k in range(_K)]
                for b in range(_B):
                    for h in range(2):
                        col = b * _C + h * 16
                        acc = ws[0] * g_v[bi, b4, pl.ds(col, 16)]
                        for k in range(1, _K):
                            acc = acc + ws[k] * g_v[bi, b4 + k, pl.ds(col, 16)]
                        o_v[bi, b, r, pl.ds(h * 16, 16)] = acc

    def tbody(p, carry):
        for t in range(_NBUF):
            s = p * _NBUF + t
            bi = t
            rbase = base0 + s * _CHUNK
            rbase_a = pl.multiple_of(rbase, _CHUNK)
            pltpu.make_async_copy(
                xt.at[pl.ds(0, _G)], g_v.at[bi], gsem.at[bi]
            ).wait()
            @pl.when(s >= _NBUF)
            def _():
                for b in range(_B):
                    pltpu.make_async_copy(
                        o_v.at[bi, b], out.at[b, pl.ds(rbase_a, _CHUNK)],
                        osem.at[bi],
                    ).wait()

            @pl.when(s + 2 < _NSTEP)
            def _():
                start_gather(s + 2, (t + 2) % _NBUF)

            compute_step(s, bi)

            for b in range(_B):
                pltpu.async_copy(
                    o_v.at[bi, b], out.at[b, pl.ds(rbase_a, _CHUNK)], osem.at[bi]
                )
        return carry

    lax.fori_loop(0, _NSTEP // _NBUF, tbody, 0)

    for sl in range(_NSTEP - _NBUF, _NSTEP):
        bi = sl % _NBUF
        rb_a = pl.multiple_of(base0 + sl * _CHUNK, _CHUNK)
        for b in range(_B):
            pltpu.make_async_copy(
                o_v.at[bi, b], out.at[b, pl.ds(rb_a, _CHUNK)], osem.at[bi]
            ).wait()


_upsample = functools.partial(
    pl.kernel,
    out_type=jax.ShapeDtypeStruct((_B, _N_OUT, _C), jnp.float32),
    mesh=plsc.VectorSubcoreMesh(core_axis_name="c", subcore_axis_name="s"),
    scratch_types=[
        pltpu.VMEM((_IROWS, _G), jnp.int32),
        pltpu.VMEM((_RPW * _K,), jnp.float32),
        pltpu.VMEM((_NBUF, _G, _BC), jnp.float32),
        pltpu.VMEM((_NBUF, _B, _CHUNK, _C), jnp.float32),
        pltpu.SemaphoreType.DMA((_NBUF,)),
        pltpu.SemaphoreType.DMA((_NBUF,)),
    ],
)(_sc_body)


def kernel(x, connection_indices, interpolation_weights):
    xt = jnp.transpose(x, (1, 0, 2)).reshape(_N_IN, _BC)
    ci2 = connection_indices.reshape(_N_OUT * _K // _G, _G)
    wf = interpolation_weights.reshape(-1)
    return _upsample(xt, ci2, wf)
